# baseline (speedup 1.0000x reference)
import functools

import jax
import jax.numpy as jnp
from jax import lax
from jax.experimental import pallas as pl
from jax.experimental.pallas import tpu as pltpu
from jax.experimental.pallas import tpu_sc as plsc

try:
    _INFO = plsc.get_sparse_core_info()
    _NC, _NS, _L = _INFO.num_cores, _INFO.num_subcores, _INFO.num_lanes
except Exception:
    _NC, _NS, _L = 2, 16, 16
_NW = _NC * _NS


def _make_sc_kernel(p):
    assert p % (_NW * _L) == 0
    n_per = p // _NW
    mesh = plsc.VectorSubcoreMesh(core_axis_name="c", subcore_axis_name="s")

    @functools.partial(
        pl.kernel,
        mesh=mesh,
        out_type=jax.ShapeDtypeStruct((p,), jnp.float32),
        scratch_types=[
            pltpu.VMEM((n_per,), jnp.float32),
            pltpu.VMEM((n_per,), jnp.float32),
            pltpu.VMEM((n_per,), jnp.float32),
            pltpu.VMEM((_L,), jnp.float32),
        ],
    )
    def sc_kernel(u_hbm, x_hbm, w_hbm, out_hbm, u_v, x_v, o_v, w_v):
        wid = lax.axis_index("s") * _NC + lax.axis_index("c")
        base = wid * n_per
        pltpu.sync_copy(u_hbm.at[pl.ds(base, n_per)], u_v)
        pltpu.sync_copy(x_hbm.at[pl.ds(base, n_per)], x_v)
        pltpu.sync_copy(w_hbm, w_v)
        w = w_v[...]
        zero = jnp.zeros((_L,), jnp.float32)
        for i in range(n_per // _L):
            sl = pl.ds(i * _L, _L)
            u = u_v[sl]
            x = x_v[sl]
            b = u - x
            o_v[sl] = x + jnp.sign(b) * jnp.maximum(jnp.abs(b) - w, zero)
        pltpu.sync_copy(o_v, out_hbm.at[pl.ds(base, n_per)])

    return sc_kernel


def kernel(u, x, weights):
    p = u.shape[0]
    w16 = jnp.full((_L,), weights, dtype=jnp.float32)
    out = _make_sc_kernel(p)(u, x, w16)
    return out.reshape(x.shape)


# speedup vs baseline: 1.0412x; 1.0412x over previous
import functools

import jax
import jax.numpy as jnp
from jax import lax
from jax.experimental import pallas as pl
from jax.experimental.pallas import tpu as pltpu
from jax.experimental.pallas import tpu_sc as plsc

try:
    _INFO = plsc.get_sparse_core_info()
    _NC, _NS, _L = _INFO.num_cores, _INFO.num_subcores, _INFO.num_lanes
except Exception:
    _NC, _NS, _L = 2, 16, 16
_NW = _NC * _NS


def _make_sc_kernel(p):
    assert p % (_NW * _L) == 0
    n_per = p // _NW
    mesh = plsc.VectorSubcoreMesh(core_axis_name="c", subcore_axis_name="s")

    @functools.partial(
        pl.kernel,
        mesh=mesh,
        out_type=jax.ShapeDtypeStruct((p,), jnp.float32),
        scratch_types=[
            pltpu.VMEM((n_per,), jnp.float32),
            pltpu.VMEM((n_per,), jnp.float32),
            pltpu.VMEM((_L,), jnp.float32),
            pltpu.SemaphoreType.DMA,
        ],
    )
    def sc_kernel(u_hbm, x_hbm, w_hbm, out_hbm, u_v, x_v, w_v, sem):
        wid = lax.axis_index("s") * _NC + lax.axis_index("c")
        base = wid * n_per
        sl_hbm = pl.ds(base, n_per)
        cp_u = pltpu.async_copy(u_hbm.at[sl_hbm], u_v, sem)
        cp_x = pltpu.async_copy(x_hbm.at[sl_hbm], x_v, sem)
        cp_w = pltpu.async_copy(w_hbm, w_v, sem)
        cp_u.wait()
        cp_x.wait()
        cp_w.wait()
        w = w_v[...]
        neg_w = -w
        for i in range(n_per // _L):
            sl = pl.ds(i * _L, _L)
            u = u_v[sl]
            b = u - x_v[sl]
            u_v[sl] = u - jnp.minimum(jnp.maximum(b, neg_w), w)
        pltpu.sync_copy(u_v, out_hbm.at[sl_hbm])

    return sc_kernel


def kernel(u, x, weights):
    p = u.shape[0]
    w16 = jnp.full((_L,), weights, dtype=jnp.float32)
    out = _make_sc_kernel(p)(u, x, w16)
    return out.reshape(x.shape)
